# baseline (speedup 1.0000x reference)
import jax
import jax.numpy as jnp
from jax.experimental import pallas as pl

_B, _D = 16384, 64
_RP = 8192
_BLK = 1024
_EPS = 1e-12


def _body(gu_ref, gi_ref, gis_ref, ones_ref, xui_ref, guo_ref, gio_ref,
          giso_ref):
    gu = gu_ref[...]
    gi = gi_ref[...]
    gis = gis_ref[...]
    guo_ref[...] = gu
    gio_ref[...] = gi
    giso_ref[...] = gis
    ones = ones_ref[...]
    c = jax.lax.dot_general(
        gis * gis, ones, (((1,), (0,)), ((), ())),
        precision=jax.lax.Precision.HIGHEST,
        preferred_element_type=jnp.float32)
    inv = 1.0 / jnp.maximum(jnp.sqrt(c), _EPS)
    lane = jax.lax.broadcasted_iota(jnp.int32, (_BLK, 128), 1)
    inv_full = jnp.where(lane < _D, inv[:, 0:1], inv[:, 1:2])
    p = gu * (gi + gis * inv_full)
    xui_ref[...] = jax.lax.dot_general(
        p, ones, (((1,), (0,)), ((), ())),
        precision=jax.lax.Precision.HIGHEST,
        preferred_element_type=jnp.float32)


def kernel(gu, gi, gis):
    gu2 = gu.reshape(_RP, 128)
    gi2 = gi.reshape(_RP, 128)
    gis2 = gis.reshape(_RP, 128)
    lane = jax.lax.broadcasted_iota(jnp.int32, (128, 2), 0)
    col = jax.lax.broadcasted_iota(jnp.int32, (128, 2), 1)
    ones = jnp.where((lane // _D) == col, 1.0, 0.0).astype(jnp.float32)

    grid = (_RP // _BLK,)
    in_spec = pl.BlockSpec((_BLK, 128), lambda i: (i, 0))
    ones_spec = pl.BlockSpec((128, 2), lambda i: (0, 0))
    xui_spec = pl.BlockSpec((_BLK, 2), lambda i: (i, 0))
    xui2, guo, gio, giso = pl.pallas_call(
        _body,
        grid=grid,
        in_specs=[in_spec, in_spec, in_spec, ones_spec],
        out_specs=(xui_spec, in_spec, in_spec, in_spec),
        out_shape=(
            jax.ShapeDtypeStruct((_RP, 2), jnp.float32),
            jax.ShapeDtypeStruct((_RP, 128), jnp.float32),
            jax.ShapeDtypeStruct((_RP, 128), jnp.float32),
            jax.ShapeDtypeStruct((_RP, 128), jnp.float32),
        ),
    )(gu2, gi2, gis2, ones)
    return (xui2.reshape(_B), guo.reshape(_B, _D), gio.reshape(_B, _D),
            giso.reshape(_B, _D))


# speedup vs baseline: 1.3076x; 1.3076x over previous
import jax
import jax.numpy as jnp
from jax.experimental import pallas as pl

_B, _D = 16384, 64
_RP = 8192
_BLK = 1024
_EPS = 1e-12


def _body(gu_ref, gi_ref, gis_ref, ones_ref, xui_ref):
    gu = gu_ref[...]
    gi = gi_ref[...]
    gis = gis_ref[...]
    ones = ones_ref[...]
    c = jax.lax.dot_general(
        gis * gis, ones, (((1,), (0,)), ((), ())),
        precision=jax.lax.Precision.HIGHEST,
        preferred_element_type=jnp.float32)
    inv = 1.0 / jnp.maximum(jnp.sqrt(c), _EPS)
    lane = jax.lax.broadcasted_iota(jnp.int32, (_BLK, 128), 1)
    inv_full = jnp.where(lane < _D, inv[:, 0:1], inv[:, 1:2])
    p = gu * (gi + gis * inv_full)
    xui_ref[...] = jax.lax.dot_general(
        p, ones, (((1,), (0,)), ((), ())),
        precision=jax.lax.Precision.HIGHEST,
        preferred_element_type=jnp.float32)


def kernel(gu, gi, gis):
    gu2 = gu.reshape(_RP, 128)
    gi2 = gi.reshape(_RP, 128)
    gis2 = gis.reshape(_RP, 128)
    lane = jax.lax.broadcasted_iota(jnp.int32, (128, 2), 0)
    col = jax.lax.broadcasted_iota(jnp.int32, (128, 2), 1)
    ones = jnp.where((lane // _D) == col, 1.0, 0.0).astype(jnp.float32)

    grid = (_RP // _BLK,)
    in_spec = pl.BlockSpec((_BLK, 128), lambda i: (i, 0))
    ones_spec = pl.BlockSpec((128, 2), lambda i: (0, 0))
    xui_spec = pl.BlockSpec((_BLK, 2), lambda i: (i, 0))
    xui2 = pl.pallas_call(
        _body,
        grid=grid,
        in_specs=[in_spec, in_spec, in_spec, ones_spec],
        out_specs=xui_spec,
        out_shape=jax.ShapeDtypeStruct((_RP, 2), jnp.float32),
    )(gu2, gi2, gis2, ones)
    return (xui2.reshape(_B), gu + 0.0, gi + 0.0, gis + 0.0)
